# baseline (speedup 1.0000x reference)
import functools

import jax
import jax.numpy as jnp
from jax import lax
from jax.experimental import pallas as pl
from jax.experimental.pallas import tpu as pltpu
from jax.experimental.pallas import tpu_sc as plsc

N_NODES = 10000
D_IN = 128
H = 64
N_CLASSES = 8
E_TOTAL = 320000

NCORES = 2
NSUB = 16
FSL = H // NSUB
SLICE = FSL * N_NODES
EG = E_TOTAL // NCORES
CE = 8000
NCHUNK = EG // CE
STEPS = CE // 16

_BN_SCALE = 0.9999950000374997


def _sc_edge_body(hh_hbm, src_hbm, dst_hbm, ea_hbm, out_hbm,
                  table_v, acc_v,
                  src0_v, dst0_v, ea0_v, src1_v, dst1_v, ea1_v,
                  sem0, sem1):
    c = lax.axis_index("c")
    s = lax.axis_index("s")

    bufs = ((src0_v, dst0_v, ea0_v, sem0), (src1_v, dst1_v, ea1_v, sem1))

    def _start(ci, b):
        base = c * EG + ci * CE
        sb, db, eb, sem = bufs[b]
        pltpu.async_copy(src_hbm.at[pl.ds(base, CE)], sb, sem)
        pltpu.async_copy(dst_hbm.at[pl.ds(base, CE)], db, sem)
        pltpu.async_copy(ea_hbm.at[pl.ds(base, CE)], eb, sem)

    def _wait(ci, b):
        base = c * EG + ci * CE
        sb, db, eb, sem = bufs[b]
        pltpu.make_async_copy(src_hbm.at[pl.ds(base, CE)], sb, sem).wait()
        pltpu.make_async_copy(dst_hbm.at[pl.ds(base, CE)], db, sem).wait()
        pltpu.make_async_copy(ea_hbm.at[pl.ds(base, CE)], eb, sem).wait()

    _start(0, 0)
    _start(1, 1)

    pltpu.sync_copy(hh_hbm.at[pl.ds(s * SLICE, SLICE)], table_v)

    _zeros16 = jnp.zeros((16,), jnp.float32)

    @plsc.parallel_loop(0, SLICE // 16, 1, unroll=8)
    def _zero(i):
        acc_v[pl.ds(i * 16, 16)] = _zeros16

    tabs = [table_v.at[pl.ds(j * N_NODES, N_NODES)] for j in range(FSL)]
    accs = [acc_v.at[pl.ds(j * N_NODES, N_NODES)] for j in range(FSL)]

    def _process(b):
        sb, db, eb, _ = bufs[b]

        @plsc.parallel_loop(0, CE, 16, unroll=1)
        def _step(off):
            sv = sb[pl.ds(off, 16)]
            dv = db[pl.ds(off, 16)]
            av = eb[pl.ds(off, 16)]
            for j in range(FSL):
                vals = plsc.load_gather(tabs[j], [sv])
                plsc.addupdate_scatter(accs[j], [dv], vals * av)

    def _outer(k, carry):
        for b in range(2):
            ci = 2 * k + b
            _wait(ci, b)
            _process(b)

            @pl.when(ci + 2 < NCHUNK)
            def _():
                _start(ci + 2, b)
        return carry
    lax.fori_loop(0, NCHUNK // 2, _outer, 0)

    pltpu.sync_copy(acc_v, out_hbm.at[pl.ds((c * NSUB + s) * SLICE, SLICE)])


@functools.cache
def _sc_edge_kernel():
    return pl.kernel(
        _sc_edge_body,
        out_type=jax.ShapeDtypeStruct((NCORES * H * N_NODES,), jnp.float32),
        mesh=plsc.VectorSubcoreMesh(core_axis_name="c", subcore_axis_name="s",
                                    num_cores=NCORES, num_subcores=NSUB),
        compiler_params=pltpu.CompilerParams(needs_layout_passes=False,
                                             use_tc_tiling_on_sc=False),
        scratch_types=[
            pltpu.VMEM((SLICE,), jnp.float32),
            pltpu.VMEM((SLICE,), jnp.float32),
            pltpu.VMEM((CE,), jnp.int32),
            pltpu.VMEM((CE,), jnp.int32),
            pltpu.VMEM((CE,), jnp.float32),
            pltpu.VMEM((CE,), jnp.int32),
            pltpu.VMEM((CE,), jnp.int32),
            pltpu.VMEM((CE,), jnp.float32),
            pltpu.SemaphoreType.DMA,
            pltpu.SemaphoreType.DMA,
        ],
    )


def _sc_edge(*args):
    return _sc_edge_kernel()(*args)


_DN0 = (((0,), (1,)), ((), ()))
_DNF = (((0,), (0,)), ((), ()))


def _tc_in_body(x_ref, w_ref, out_ref):
    out_ref[...] = lax.dot_general(w_ref[...], x_ref[...], _DN0,
                                   preferred_element_type=jnp.float32)


def _post_layer(acc_ref, b_ref, g_ref, be_ref):
    h = acc_ref[0] + acc_ref[1] + b_ref[...]
    h = jnp.maximum(h, 0.0)
    return h * (g_ref[...] * _BN_SCALE) + be_ref[...]


def _tc_mid_body(acc_ref, b_ref, g_ref, be_ref, w_ref, out_ref):
    h = _post_layer(acc_ref, b_ref, g_ref, be_ref)
    out_ref[...] = lax.dot_general(w_ref[...], h, _DNF,
                                   preferred_element_type=jnp.float32)


def _gelu(z):
    return 0.5 * z * (1.0 + lax.erf(z * 0.7071067811865476))


def _tc_out_body(acc_ref, b_ref, g_ref, be_ref,
                 C1_ref, c1_ref, C2_ref, c2_ref, C3_ref, c3_ref, out_ref):
    h = _post_layer(acc_ref, b_ref, g_ref, be_ref)
    z = lax.dot_general(C1_ref[...], h, _DNF,
                        preferred_element_type=jnp.float32) + c1_ref[...]
    z = _gelu(z)
    z = lax.dot_general(C2_ref[...], z, _DNF,
                        preferred_element_type=jnp.float32) + c2_ref[...]
    z = _gelu(z)
    out_ref[...] = lax.dot_general(C3_ref[...], z, _DNF,
                                   preferred_element_type=jnp.float32) + c3_ref[...]


def _tc_in(x, w):
    return pl.pallas_call(
        _tc_in_body,
        out_shape=jax.ShapeDtypeStruct((H, N_NODES), jnp.float32),
    )(x, w)


def _tc_mid(acc, b, g, be, w):
    return pl.pallas_call(
        _tc_mid_body,
        out_shape=jax.ShapeDtypeStruct((H, N_NODES), jnp.float32),
    )(acc, b, g, be, w)


def _tc_out(acc, b, g, be, C1, c1, C2, c2, C3, c3):
    return pl.pallas_call(
        _tc_out_body,
        out_shape=jax.ShapeDtypeStruct((N_CLASSES, N_NODES), jnp.float32),
    )(acc, b, g, be, C1, c1, C2, c2, C3, c3)


def _col(v):
    return v.reshape(-1, 1)


def kernel(x, edge_index, edge_attr, W0, b0, W1, b1, W2, b2,
           bn_g0, bn_b0, bn_g1, bn_b1, bn_g2, bn_b2,
           C1, c1, C2, c2, C3, c3):
    src = edge_index[0]
    dst = edge_index[1]

    hh = _tc_in(x, W0)
    acc = _sc_edge(hh.reshape(-1), src, dst, edge_attr).reshape(NCORES, H, N_NODES)
    hh = _tc_mid(acc, _col(b0), _col(bn_g0), _col(bn_b0), W1)
    acc = _sc_edge(hh.reshape(-1), src, dst, edge_attr).reshape(NCORES, H, N_NODES)
    hh = _tc_mid(acc, _col(b1), _col(bn_g1), _col(bn_b1), W2)
    acc = _sc_edge(hh.reshape(-1), src, dst, edge_attr).reshape(NCORES, H, N_NODES)
    outT = _tc_out(acc, _col(b2), _col(bn_g2), _col(bn_b2),
                   C1, _col(c1), C2, _col(c2), C3, _col(c3))
    return outT.T


# speedup vs baseline: 1.0662x; 1.0662x over previous
import functools

import jax
import jax.numpy as jnp
from jax import lax
from jax.experimental import pallas as pl
from jax.experimental.pallas import tpu as pltpu
from jax.experimental.pallas import tpu_sc as plsc

N_NODES = 10000
D_IN = 128
H = 64
N_CLASSES = 8
E_TOTAL = 320000

NCORES = 2
NSUB = 16
FSL = H // NSUB
SLICE = FSL * N_NODES
EG = E_TOTAL // NCORES
CE = 4000
NCHUNK = EG // CE
STEPS = CE // 16

_BN_SCALE = 0.9999950000374997


def _sc_edge_body(hh_hbm, src_hbm, dst_hbm, ea_hbm, out_hbm,
                  table_v, acc_v,
                  src0_v, dst0_v, ea0_v, src1_v, dst1_v, ea1_v,
                  sem0, sem1):
    c = lax.axis_index("c")
    s = lax.axis_index("s")

    bufs = ((src0_v, dst0_v, ea0_v, sem0), (src1_v, dst1_v, ea1_v, sem1))

    def _start(ci, b):
        base = c * EG + ci * CE
        sb, db, eb, sem = bufs[b]
        pltpu.async_copy(src_hbm.at[pl.ds(base, CE)], sb, sem)
        pltpu.async_copy(dst_hbm.at[pl.ds(base, CE)], db, sem)
        pltpu.async_copy(ea_hbm.at[pl.ds(base, CE)], eb, sem)

    def _wait(ci, b):
        base = c * EG + ci * CE
        sb, db, eb, sem = bufs[b]
        pltpu.make_async_copy(src_hbm.at[pl.ds(base, CE)], sb, sem).wait()
        pltpu.make_async_copy(dst_hbm.at[pl.ds(base, CE)], db, sem).wait()
        pltpu.make_async_copy(ea_hbm.at[pl.ds(base, CE)], eb, sem).wait()

    _start(0, 0)
    _start(1, 1)

    pltpu.sync_copy(hh_hbm.at[pl.ds(s * SLICE, SLICE)], table_v)

    _zeros16 = jnp.zeros((16,), jnp.float32)

    @plsc.parallel_loop(0, SLICE // 16, 1, unroll=8)
    def _zero(i):
        acc_v[pl.ds(i * 16, 16)] = _zeros16

    tabs = [table_v.at[pl.ds(j * N_NODES, N_NODES)] for j in range(FSL)]
    accs = [acc_v.at[pl.ds(j * N_NODES, N_NODES)] for j in range(FSL)]

    def _process(b):
        sb, db, eb, _ = bufs[b]

        @plsc.parallel_loop(0, CE, 16, unroll=2)
        def _step(off):
            sv = sb[pl.ds(off, 16)]
            dv = db[pl.ds(off, 16)]
            av = eb[pl.ds(off, 16)]
            for j in range(FSL):
                vals = plsc.load_gather(tabs[j], [sv])
                plsc.addupdate_scatter(accs[j], [dv], vals * av)

    def _outer(k, carry):
        for b in range(2):
            ci = 2 * k + b
            _wait(ci, b)
            _process(b)

            @pl.when(ci + 2 < NCHUNK)
            def _():
                _start(ci + 2, b)
        return carry
    lax.fori_loop(0, NCHUNK // 2, _outer, 0)

    pltpu.sync_copy(acc_v, out_hbm.at[pl.ds((c * NSUB + s) * SLICE, SLICE)])


@functools.cache
def _sc_edge_kernel():
    return pl.kernel(
        _sc_edge_body,
        out_type=jax.ShapeDtypeStruct((NCORES * H * N_NODES,), jnp.float32),
        mesh=plsc.VectorSubcoreMesh(core_axis_name="c", subcore_axis_name="s",
                                    num_cores=NCORES, num_subcores=NSUB),
        compiler_params=pltpu.CompilerParams(needs_layout_passes=False,
                                             use_tc_tiling_on_sc=False),
        scratch_types=[
            pltpu.VMEM((SLICE,), jnp.float32),
            pltpu.VMEM((SLICE,), jnp.float32),
            pltpu.VMEM((CE,), jnp.int32),
            pltpu.VMEM((CE,), jnp.int32),
            pltpu.VMEM((CE,), jnp.float32),
            pltpu.VMEM((CE,), jnp.int32),
            pltpu.VMEM((CE,), jnp.int32),
            pltpu.VMEM((CE,), jnp.float32),
            pltpu.SemaphoreType.DMA,
            pltpu.SemaphoreType.DMA,
        ],
    )


def _sc_edge(*args):
    return _sc_edge_kernel()(*args)


_DN0 = (((0,), (1,)), ((), ()))
_DNF = (((0,), (0,)), ((), ()))


def _tc_in_body(x_ref, w_ref, out_ref):
    out_ref[...] = lax.dot_general(w_ref[...], x_ref[...], _DN0,
                                   preferred_element_type=jnp.float32)


def _post_layer(acc_ref, b_ref, g_ref, be_ref):
    h = acc_ref[0] + acc_ref[1] + b_ref[...]
    h = jnp.maximum(h, 0.0)
    return h * (g_ref[...] * _BN_SCALE) + be_ref[...]


def _tc_mid_body(acc_ref, b_ref, g_ref, be_ref, w_ref, out_ref):
    h = _post_layer(acc_ref, b_ref, g_ref, be_ref)
    out_ref[...] = lax.dot_general(w_ref[...], h, _DNF,
                                   preferred_element_type=jnp.float32)


def _gelu(z):
    return 0.5 * z * (1.0 + lax.erf(z * 0.7071067811865476))


def _tc_out_body(acc_ref, b_ref, g_ref, be_ref,
                 C1_ref, c1_ref, C2_ref, c2_ref, C3_ref, c3_ref, out_ref):
    h = _post_layer(acc_ref, b_ref, g_ref, be_ref)
    z = lax.dot_general(C1_ref[...], h, _DNF,
                        preferred_element_type=jnp.float32) + c1_ref[...]
    z = _gelu(z)
    z = lax.dot_general(C2_ref[...], z, _DNF,
                        preferred_element_type=jnp.float32) + c2_ref[...]
    z = _gelu(z)
    out_ref[...] = lax.dot_general(C3_ref[...], z, _DNF,
                                   preferred_element_type=jnp.float32) + c3_ref[...]


def _tc_in(x, w):
    return pl.pallas_call(
        _tc_in_body,
        out_shape=jax.ShapeDtypeStruct((H, N_NODES), jnp.float32),
    )(x, w)


def _tc_mid(acc, b, g, be, w):
    return pl.pallas_call(
        _tc_mid_body,
        out_shape=jax.ShapeDtypeStruct((H, N_NODES), jnp.float32),
    )(acc, b, g, be, w)


def _tc_out(acc, b, g, be, C1, c1, C2, c2, C3, c3):
    return pl.pallas_call(
        _tc_out_body,
        out_shape=jax.ShapeDtypeStruct((N_CLASSES, N_NODES), jnp.float32),
    )(acc, b, g, be, C1, c1, C2, c2, C3, c3)


def _col(v):
    return v.reshape(-1, 1)


def kernel(x, edge_index, edge_attr, W0, b0, W1, b1, W2, b2,
           bn_g0, bn_b0, bn_g1, bn_b1, bn_g2, bn_b2,
           C1, c1, C2, c2, C3, c3):
    src = edge_index[0]
    dst = edge_index[1]

    hh = _tc_in(x, W0)
    acc = _sc_edge(hh.reshape(-1), src, dst, edge_attr).reshape(NCORES, H, N_NODES)
    hh = _tc_mid(acc, _col(b0), _col(bn_g0), _col(bn_b0), W1)
    acc = _sc_edge(hh.reshape(-1), src, dst, edge_attr).reshape(NCORES, H, N_NODES)
    hh = _tc_mid(acc, _col(b1), _col(bn_g1), _col(bn_b1), W2)
    acc = _sc_edge(hh.reshape(-1), src, dst, edge_attr).reshape(NCORES, H, N_NODES)
    outT = _tc_out(acc, _col(b2), _col(bn_g2), _col(bn_b2),
                   C1, _col(c1), C2, _col(c2), C3, _col(c3))
    return outT.T
